# Initial kernel scaffold; baseline (speedup 1.0000x reference)
import jax
import jax.numpy as jnp
from jax.experimental import pallas as pl


def kernel(points, features, edge_index, W_c, b_c, W_n, b_n):
    raise NotImplementedError("write your pallas kernel here")


# speedup vs baseline: 2.5684x; 2.5684x over previous
import functools

import jax
import jax.numpy as jnp
from jax import lax
from jax.experimental import pallas as pl
from jax.experimental.pallas import tpu as pltpu
from jax.experimental.pallas import tpu_sc as plsc

N = 10000
D = 128
K = 16
E_MESH = 160000
E_TOT = E_MESH + N * K

NP = 10240
R = 128
C = 1024
NCH = NP // C
BIG = 1e30

NC, NS = 2, 16
EPW = E_TOT // (NC * NS)
CHUNK = 80
NCHUNK = EPW // CHUNK
CL = 16
NA = 10240
RPS = NA // NS


def _knn_body(ptsr_ref, ptst_ref, idx_ref, d_ref):
    pr = ptsr_ref[...]
    sq_r = jnp.sum(pr * pr, axis=1, keepdims=True)
    row0 = pl.program_id(0) * R
    rowg = row0 + lax.broadcasted_iota(jnp.int32, (R, C), 0)
    citer = lax.broadcasted_iota(jnp.int32, (R, C), 1)

    def chunk_dist(j, _):
        pt = ptst_ref[j]
        sq_c = jnp.sum(pt * pt, axis=0, keepdims=True)
        dot = jnp.dot(pr, pt, preferred_element_type=jnp.float32)
        d = (sq_r - 2.0 * dot) + sq_c
        col = j * C + citer
        d_ref[j] = jnp.where((col == rowg) | (col >= N), BIG, d)
        return 0

    lax.fori_loop(0, NCH, chunk_dist, 0)

    for r in range(K):
        def chunk_min(j, m):
            return jnp.minimum(m, jnp.min(d_ref[j], axis=1, keepdims=True))

        m = lax.fori_loop(0, NCH, chunk_min,
                          jnp.full((R, 1), jnp.inf, jnp.float32))

        def chunk_idx(j, i):
            col = j * C + citer
            cand = jnp.min(jnp.where(d_ref[j] == m, col, jnp.int32(NP)),
                           axis=1, keepdims=True)
            return jnp.minimum(i, cand)

        idx = lax.fori_loop(0, NCH, chunk_idx,
                            jnp.full((R, 1), NP, jnp.int32))

        def chunk_upd(j, _):
            col = j * C + citer
            d_ref[j] = jnp.where(col == idx, BIG, d_ref[j])
            return 0

        lax.fori_loop(0, NCH, chunk_upd, 0)
        idx_ref[:, r:r + 1] = idx


def _knn_call(pts_pad, pts_tc):
    return pl.pallas_call(
        _knn_body,
        grid=(NP // R,),
        in_specs=[
            pl.BlockSpec((R, 8), lambda i: (i, 0)),
            pl.BlockSpec((NCH, 8, C), lambda i: (0, 0, 0)),
        ],
        out_specs=pl.BlockSpec((R, K), lambda i: (i, 0)),
        out_shape=jax.ShapeDtypeStruct((NP, K), jnp.int32),
        scratch_shapes=[pltpu.VMEM((NCH, R, C), jnp.float32)],
    )(pts_pad, pts_tc)


def _sc_body(src_hbm, dst_hbm, feat_hbm, zf_hbm, ones_hbm, accf_hbm, accc_hbm,
             src_v, dst_v, rows_v, ones_v, accf_sh, sem):
    c = lax.axis_index("c")
    s = lax.axis_index("s")
    base = c * (E_TOT // NC) + s * EPW
    pltpu.sync_copy(zf_hbm.at[pl.ds(s * RPS, RPS)],
                    accf_sh.at[pl.ds(s * RPS, RPS)])
    pltpu.sync_copy(ones_hbm, ones_v)
    plsc.subcore_barrier()

    @pl.loop(0, NCHUNK)
    def _feat_chunk(k):
        off = base + k * CHUNK
        pltpu.sync_copy(src_hbm.at[pl.ds(off, CHUNK)], src_v)
        pltpu.sync_copy(dst_hbm.at[pl.ds(off, CHUNK)], dst_v.at[0])
        pltpu.async_copy(feat_hbm.at[src_v], rows_v, sem).wait()
        pltpu.sync_copy(rows_v, accf_sh.at[dst_v.at[0]], add=True)

    plsc.subcore_barrier()
    pltpu.sync_copy(accf_sh.at[pl.ds(s * RPS, RPS)],
                    accf_hbm.at[c].at[pl.ds(s * RPS, RPS)])
    pltpu.sync_copy(zf_hbm.at[pl.ds(s * RPS, RPS)],
                    accf_sh.at[pl.ds(s * RPS, RPS)])
    plsc.subcore_barrier()

    @pl.loop(0, NCHUNK)
    def _cnt_chunk(k):
        off = base + k * CHUNK
        pltpu.sync_copy(dst_hbm.at[pl.ds(off, CHUNK)], dst_v.at[0])
        pltpu.sync_copy(ones_v, accf_sh.at[dst_v.at[0]], add=True)

    plsc.subcore_barrier()
    pltpu.sync_copy(accf_sh.at[pl.ds(s * RPS, RPS)],
                    accc_hbm.at[c].at[pl.ds(s * RPS, RPS)])


@functools.cache
def _sc_aggregate_fn():
    mesh = plsc.VectorSubcoreMesh(core_axis_name="c", subcore_axis_name="s",
                                  num_cores=NC, num_subcores=NS)
    return pl.kernel(
        _sc_body,
        out_type=(
            jax.ShapeDtypeStruct((NC, NA, D), jnp.float32),
            jax.ShapeDtypeStruct((NC, NA, D), jnp.float32),
        ),
        mesh=mesh,
        scratch_types=[
            pltpu.VMEM((CHUNK,), jnp.int32),
            pltpu.VMEM((1, CHUNK), jnp.int32),
            pltpu.VMEM((CHUNK, D), jnp.float32),
            pltpu.VMEM((CHUNK, D), jnp.float32),
            pltpu.VMEM_SHARED((NA, D), jnp.float32),
            pltpu.SemaphoreType.DMA,
        ],
    )


TB = 1000


def _final_body(f_ref, a0_ref, a1_ref, c0_ref, c1_ref, wc_ref, wn_ref,
                b_ref, o_ref):
    f = f_ref[...]
    sumf = a0_ref[0] + a1_ref[0]
    cnt = c0_ref[0][:, 0:1] + c1_ref[0][:, 0:1]
    agg = sumf / cnt
    acc = jnp.dot(f, wc_ref[...], preferred_element_type=jnp.float32)
    acc = acc + jnp.dot(agg, wn_ref[...], preferred_element_type=jnp.float32)
    o_ref[...] = jnp.maximum(acc + b_ref[...], 0.0)


def _final_call(features, accf, accc, W_c, W_n, bias):
    return pl.pallas_call(
        _final_body,
        grid=(N // TB,),
        in_specs=[
            pl.BlockSpec((TB, D), lambda i: (i, 0)),
            pl.BlockSpec((1, TB, D), lambda i: (0, i, 0)),
            pl.BlockSpec((1, TB, D), lambda i: (1, i, 0)),
            pl.BlockSpec((1, TB, D), lambda i: (0, i, 0)),
            pl.BlockSpec((1, TB, D), lambda i: (1, i, 0)),
            pl.BlockSpec((D, D), lambda i: (0, 0)),
            pl.BlockSpec((D, D), lambda i: (0, 0)),
            pl.BlockSpec((1, D), lambda i: (0, 0)),
        ],
        out_specs=pl.BlockSpec((TB, D), lambda i: (i, 0)),
        out_shape=jax.ShapeDtypeStruct((N, D), jnp.float32),
    )(features, accf, accf, accc, accc, W_c, W_n, bias)


def kernel(points, features, edge_index, W_c, b_c, W_n, b_n):
    pts_pad = jnp.zeros((NP, 8), jnp.float32).at[:N, :3].set(points)
    pts_tc = pts_pad.T.reshape(8, NCH, C).transpose(1, 0, 2)
    nbr = _knn_call(pts_pad, pts_tc)
    knn_src = nbr[:N].reshape(-1)
    knn_dst = jnp.broadcast_to(
        jnp.arange(N, dtype=jnp.int32)[:, None], (N, K)).reshape(-1)
    esrc = jnp.concatenate([edge_index[0].astype(jnp.int32), knn_src])
    edst = jnp.concatenate([edge_index[1].astype(jnp.int32), knn_dst])
    zf = jnp.zeros((NA, D), jnp.float32)
    ones = jnp.ones((CHUNK, D), jnp.float32)
    accf, accc = _sc_aggregate_fn()(esrc, edst, features, zf, ones)
    bias = (b_c + b_n)[None, :]
    return _final_call(features, accf, accc, W_c, W_n, bias)
